# Initial kernel scaffold; baseline (speedup 1.0000x reference)
import jax
import jax.numpy as jnp
from jax.experimental import pallas as pl


def kernel(x, edge_index, W1, b1, W2, b2):
    raise NotImplementedError("write your pallas kernel here")


# speedup vs baseline: 23.0269x; 23.0269x over previous
import functools

import jax
import jax.numpy as jnp
from jax import lax
from jax.experimental import pallas as pl
from jax.experimental.pallas import tpu as pltpu
from jax.experimental.pallas import tpu_sc as plsc

N = 10000
D = 128
E = 320000

NC = 2
NS = 16
NW = NC * NS

NP = 10240
RT = NP // NS
K = 128
CH = (E + NW * K - 1) // (NW * K)
EPAD = NW * CH * K

R = 1024
G = NP // R

_mesh = plsc.VectorSubcoreMesh(
    core_axis_name="c", subcore_axis_name="s", num_cores=NC, num_subcores=NS
)


@functools.partial(
    pl.kernel,
    out_type=jax.ShapeDtypeStruct((NC, NP), jnp.float32),
    mesh=_mesh,
    scratch_types=[
        pltpu.VMEM((CH, K), jnp.int32),
        pltpu.VMEM((K,), jnp.float32),
        pltpu.VMEM((RT,), jnp.float32),
        pltpu.VMEM_SHARED((NP,), jnp.float32),
    ],
)
def _deg_kernel(dst_hbm, out_hbm, idx_v, ones_v, zrow_v, deg_sh):
    c = lax.axis_index("c")
    s = lax.axis_index("s")
    wid = s * NC + c
    r0 = s * RT

    pltpu.sync_copy(dst_hbm.at[wid], idx_v)
    for k in range(K // 16):
        ones_v[pl.ds(k * 16, 16)] = jnp.ones((16,), jnp.float32)
    for k in range(RT // 16):
        zrow_v[pl.ds(k * 16, 16)] = jnp.zeros((16,), jnp.float32)
    pltpu.sync_copy(zrow_v, deg_sh.at[pl.ds(r0, RT)])
    plsc.subcore_barrier()

    def body(j, carry):
        pltpu.sync_copy(ones_v, deg_sh.at[idx_v.at[j]], add=True)
        return carry

    lax.fori_loop(0, CH, body, 0, unroll=False)
    plsc.subcore_barrier()
    pltpu.sync_copy(deg_sh.at[pl.ds(r0, RT)], out_hbm.at[c, pl.ds(r0, RT)])


@functools.partial(
    pl.kernel,
    out_type=jax.ShapeDtypeStruct((NC, NP, D), jnp.float32),
    mesh=_mesh,
    scratch_types=[
        pltpu.VMEM((CH, K), jnp.int32),
        pltpu.VMEM((CH, K), jnp.int32),
        pltpu.VMEM((K, D), jnp.float32),
        pltpu.VMEM_SHARED((NP, D), jnp.float32),
        pltpu.SemaphoreType.DMA,
    ],
)
def _agg_kernel(y_hbm, src_hbm, dst_hbm, zeros_hbm, out_hbm,
                src_v, dst_v, rows_v, agg_sh, sem):
    c = lax.axis_index("c")
    s = lax.axis_index("s")
    wid = s * NC + c
    r0 = s * RT

    pltpu.sync_copy(src_hbm.at[wid], src_v)
    pltpu.sync_copy(dst_hbm.at[wid], dst_v)

    @pl.when(c == 0)
    def _():
        pltpu.sync_copy(y_hbm.at[pl.ds(r0, RT)], agg_sh.at[pl.ds(r0, RT)])

    @pl.when(c != 0)
    def _():
        pltpu.sync_copy(zeros_hbm.at[pl.ds(r0, RT)], agg_sh.at[pl.ds(r0, RT)])

    plsc.subcore_barrier()

    def body(j, carry):
        pltpu.async_copy(y_hbm.at[src_v.at[j]], rows_v, sem).wait()
        pltpu.sync_copy(rows_v, agg_sh.at[dst_v.at[j]], add=True)
        return carry

    lax.fori_loop(0, CH, body, 0, unroll=False)
    plsc.subcore_barrier()
    pltpu.sync_copy(agg_sh.at[pl.ds(r0, RT)], out_hbm.at[c, pl.ds(r0, RT)])


def _dinv_of(deg_ref):
    deg = deg_ref[0, :] + deg_ref[1, :] + 1.0
    return lax.rsqrt(deg)[:, None]


def _tc_in_body(x_ref, w_ref, deg_ref, y_ref):
    y = jnp.dot(x_ref[...], w_ref[...], preferred_element_type=jnp.float32)
    y_ref[...] = y * _dinv_of(deg_ref)


def _tc_mid_body(p_ref, deg_ref, b_ref, w_ref, y_ref):
    dinv = _dinv_of(deg_ref)
    h = jnp.maximum((p_ref[0] + p_ref[1]) * dinv + b_ref[...][None, :], 0.0)
    y_ref[...] = jnp.dot(h, w_ref[...], preferred_element_type=jnp.float32) * dinv


def _tc_out_body(p_ref, deg_ref, b_ref, o_ref):
    dinv = _dinv_of(deg_ref)
    o_ref[...] = jnp.maximum((p_ref[0] + p_ref[1]) * dinv + b_ref[...][None, :], 0.0)


_xspec = pl.BlockSpec((R, D), lambda i: (i, 0))
_wspec = pl.BlockSpec((D, D), lambda i: (0, 0))
_dspec = pl.BlockSpec((2, R), lambda i: (0, i))
_pspec = pl.BlockSpec((2, R, D), lambda i: (0, i, 0))
_bspec = pl.BlockSpec((D,), lambda i: (0,))
_ospec = pl.BlockSpec((R, D), lambda i: (i, 0))
_oshape = jax.ShapeDtypeStruct((NP, D), jnp.float32)

_tc_in = pl.pallas_call(
    _tc_in_body, grid=(G,),
    in_specs=[_xspec, _wspec, _dspec], out_specs=_ospec, out_shape=_oshape)
_tc_mid = pl.pallas_call(
    _tc_mid_body, grid=(G,),
    in_specs=[_pspec, _dspec, _bspec, _wspec], out_specs=_ospec, out_shape=_oshape)
_tc_out = pl.pallas_call(
    _tc_out_body, grid=(G,),
    in_specs=[_pspec, _dspec, _bspec], out_specs=_ospec, out_shape=_oshape)


def kernel(x, edge_index, W1, b1, W2, b2):
    x_pad = jnp.pad(x, ((0, NP - N), (0, 0)))
    pad_idx = (N + (jnp.arange(EPAD - E, dtype=jnp.int32) % (NP - N)))
    src = jnp.concatenate([edge_index[0], pad_idx]).reshape(NW, CH, K)
    dst = jnp.concatenate([edge_index[1], pad_idx]).reshape(NW, CH, K)
    zeros2d = jnp.zeros((NP, D), jnp.float32)

    degp = _deg_kernel(dst)
    y1 = _tc_in(x_pad, W1, degp)
    p1 = _agg_kernel(y1, src, dst, zeros2d)
    y2 = _tc_mid(p1, degp, b1, W2)
    p2 = _agg_kernel(y2, src, dst, zeros2d)
    out = _tc_out(p2, degp, b2)
    return out[:N]


# speedup vs baseline: 25.1458x; 1.0920x over previous
import functools

import jax
import jax.numpy as jnp
from jax import lax
from jax.experimental import pallas as pl
from jax.experimental.pallas import tpu as pltpu
from jax.experimental.pallas import tpu_sc as plsc

N = 10000
D = 128
E = 320000

NC = 2
NS = 16
NW = NC * NS

NP = 10240
RT = NP // NS
K = 96
CH = (E + NW * K - 1) // (NW * K)
EPAD = NW * CH * K

R = 1024
G = NP // R

_mesh = plsc.VectorSubcoreMesh(
    core_axis_name="c", subcore_axis_name="s", num_cores=NC, num_subcores=NS
)


@functools.partial(
    pl.kernel,
    out_type=jax.ShapeDtypeStruct((NC, NP), jnp.float32),
    mesh=_mesh,
    scratch_types=[
        pltpu.VMEM((CH, K), jnp.int32),
        pltpu.VMEM((K,), jnp.float32),
        pltpu.VMEM((RT,), jnp.float32),
        pltpu.VMEM_SHARED((NP,), jnp.float32),
    ],
)
def _deg_kernel(dst_hbm, out_hbm, idx_v, ones_v, zrow_v, deg_sh):
    c = lax.axis_index("c")
    s = lax.axis_index("s")
    wid = s * NC + c
    r0 = s * RT

    pltpu.sync_copy(dst_hbm.at[wid], idx_v)
    for k in range(K // 16):
        ones_v[pl.ds(k * 16, 16)] = jnp.ones((16,), jnp.float32)
    for k in range(RT // 16):
        zrow_v[pl.ds(k * 16, 16)] = jnp.zeros((16,), jnp.float32)
    pltpu.sync_copy(zrow_v, deg_sh.at[pl.ds(r0, RT)])
    plsc.subcore_barrier()

    def body(j, carry):
        pltpu.sync_copy(ones_v, deg_sh.at[idx_v.at[j]], add=True)
        return carry

    lax.fori_loop(0, CH, body, 0, unroll=False)
    plsc.subcore_barrier()
    pltpu.sync_copy(deg_sh.at[pl.ds(r0, RT)], out_hbm.at[c, pl.ds(r0, RT)])


@functools.partial(
    pl.kernel,
    out_type=jax.ShapeDtypeStruct((NC, NP, D), jnp.float32),
    mesh=_mesh,
    scratch_types=[
        pltpu.VMEM((CH * K,), jnp.int32),
        pltpu.VMEM((CH, K), jnp.int32),
        pltpu.VMEM((K, D), jnp.float32),
        pltpu.VMEM((K, D), jnp.float32),
        pltpu.VMEM_SHARED((NP, D), jnp.float32),
        pltpu.SemaphoreType.DMA,
        pltpu.SemaphoreType.DMA,
    ],
)
def _agg_kernel(y_hbm, src_hbm, dst_hbm, zeros_hbm, out_hbm,
                src_v, dst_v, rows0_v, rows1_v, agg_sh, sem0, sem1):
    c = lax.axis_index("c")
    s = lax.axis_index("s")
    wid = s * NC + c
    r0 = s * RT

    pltpu.sync_copy(src_hbm.at[wid], src_v)
    pltpu.sync_copy(dst_hbm.at[wid], dst_v)

    @pl.when(c == 0)
    def _():
        pltpu.sync_copy(y_hbm.at[pl.ds(r0, RT)], agg_sh.at[pl.ds(r0, RT)])

    @pl.when(c != 0)
    def _():
        pltpu.sync_copy(zeros_hbm.at[pl.ds(r0, RT)], agg_sh.at[pl.ds(r0, RT)])

    plsc.subcore_barrier()

    def body(jj, carry):
        j = 2 * jj
        d0 = pltpu.async_copy(y_hbm.at[src_v.at[pl.ds(j * K, K)]], rows0_v, sem0)
        d1 = pltpu.async_copy(y_hbm.at[src_v.at[pl.ds((j + 1) * K, K)]], rows1_v, sem1)
        d0.wait()
        pltpu.sync_copy(rows0_v, agg_sh.at[dst_v.at[j]], add=True)
        d1.wait()
        pltpu.sync_copy(rows1_v, agg_sh.at[dst_v.at[j + 1]], add=True)
        return carry

    lax.fori_loop(0, CH // 2, body, 0, unroll=False)

    if CH % 2:
        d = pltpu.async_copy(y_hbm.at[src_v.at[pl.ds((CH - 1) * K, K)]], rows0_v, sem0)
        d.wait()
        pltpu.sync_copy(rows0_v, agg_sh.at[dst_v.at[CH - 1]], add=True)
    plsc.subcore_barrier()
    pltpu.sync_copy(agg_sh.at[pl.ds(r0, RT)], out_hbm.at[c, pl.ds(r0, RT)])


def _dinv_of(deg_ref):
    deg = deg_ref[0, :] + deg_ref[1, :] + 1.0
    return lax.rsqrt(deg)[:, None]


def _tc_in_body(x_ref, w_ref, deg_ref, y_ref):
    y = jnp.dot(x_ref[...], w_ref[...], preferred_element_type=jnp.float32)
    y_ref[...] = y * _dinv_of(deg_ref)


def _tc_mid_body(p_ref, deg_ref, b_ref, w_ref, y_ref):
    dinv = _dinv_of(deg_ref)
    h = jnp.maximum((p_ref[0] + p_ref[1]) * dinv + b_ref[...][None, :], 0.0)
    y_ref[...] = jnp.dot(h, w_ref[...], preferred_element_type=jnp.float32) * dinv


def _tc_out_body(p_ref, deg_ref, b_ref, o_ref):
    dinv = _dinv_of(deg_ref)
    o_ref[...] = jnp.maximum((p_ref[0] + p_ref[1]) * dinv + b_ref[...][None, :], 0.0)


_xspec = pl.BlockSpec((R, D), lambda i: (i, 0))
_wspec = pl.BlockSpec((D, D), lambda i: (0, 0))
_dspec = pl.BlockSpec((2, R), lambda i: (0, i))
_pspec = pl.BlockSpec((2, R, D), lambda i: (0, i, 0))
_bspec = pl.BlockSpec((D,), lambda i: (0,))
_ospec = pl.BlockSpec((R, D), lambda i: (i, 0))
_oshape = jax.ShapeDtypeStruct((NP, D), jnp.float32)

_tc_in = pl.pallas_call(
    _tc_in_body, grid=(G,),
    in_specs=[_xspec, _wspec, _dspec], out_specs=_ospec, out_shape=_oshape)
_tc_mid = pl.pallas_call(
    _tc_mid_body, grid=(G,),
    in_specs=[_pspec, _dspec, _bspec, _wspec], out_specs=_ospec, out_shape=_oshape)
_tc_out = pl.pallas_call(
    _tc_out_body, grid=(G,),
    in_specs=[_pspec, _dspec, _bspec], out_specs=_ospec, out_shape=_oshape)


def kernel(x, edge_index, W1, b1, W2, b2):
    x_pad = jnp.pad(x, ((0, NP - N), (0, 0)))
    pad_idx = (N + (jnp.arange(EPAD - E, dtype=jnp.int32) % (NP - N)))
    src = jnp.concatenate([edge_index[0], pad_idx]).reshape(NW, CH * K)
    dst = jnp.concatenate([edge_index[1], pad_idx]).reshape(NW, CH, K)
    zeros2d = jnp.zeros((NP, D), jnp.float32)

    degp = _deg_kernel(dst)
    y1 = _tc_in(x_pad, W1, degp)
    p1 = _agg_kernel(y1, src, dst, zeros2d)
    y2 = _tc_mid(p1, degp, b1, W2)
    p2 = _agg_kernel(y2, src, dst, zeros2d)
    out = _tc_out(p2, degp, b2)
    return out[:N]


# speedup vs baseline: 33.6830x; 1.3395x over previous
import functools

import jax
import jax.numpy as jnp
from jax import lax
from jax.experimental import pallas as pl
from jax.experimental.pallas import tpu as pltpu
from jax.experimental.pallas import tpu_sc as plsc

N = 10000
D = 128
E = 320000

NC = 2
NS = 16
NW = NC * NS

NP = 10240
RT = NP // NS
K = 128
CH = 80
B = 40
EPAD = NW * CH * K

R = 1024
G = NP // R

_mesh = plsc.VectorSubcoreMesh(
    core_axis_name="c", subcore_axis_name="s", num_cores=NC, num_subcores=NS
)


@functools.partial(
    pl.kernel,
    out_type=jax.ShapeDtypeStruct((NC, NP), jnp.float32),
    mesh=_mesh,
    scratch_types=[
        pltpu.VMEM((CH, K), jnp.int32),
        pltpu.VMEM((K,), jnp.float32),
        pltpu.VMEM((RT,), jnp.float32),
        pltpu.VMEM_SHARED((NP,), jnp.float32),
        pltpu.SemaphoreType.DMA,
        pltpu.SemaphoreType.DMA,
        pltpu.SemaphoreType.DMA,
    ],
)
def _deg_kernel(dst_hbm, out_hbm, idx_v, ones_v, zrow_v, deg_sh, sd0, sd1, sd2):
    c = lax.axis_index("c")
    s = lax.axis_index("s")
    wid = s * NC + c
    r0 = s * RT

    pltpu.sync_copy(dst_hbm.at[wid], idx_v)
    for k in range(K // 16):
        ones_v[pl.ds(k * 16, 16)] = jnp.ones((16,), jnp.float32)
    for k in range(RT // 16):
        zrow_v[pl.ds(k * 16, 16)] = jnp.zeros((16,), jnp.float32)
    pltpu.sync_copy(zrow_v, deg_sh.at[pl.ds(r0, RT)])
    plsc.subcore_barrier()

    sds = (sd0, sd1, sd2)

    def body(jj, carry):
        j = 3 * jj
        ds = [pltpu.async_copy(ones_v, deg_sh.at[idx_v.at[j + i]], sds[i],
                               add=True)
              for i in range(3)]
        for d in ds:
            d.wait()
        return carry

    lax.fori_loop(0, CH // 3, body, 0, unroll=False)
    for i in range(CH % 3):
        pltpu.sync_copy(ones_v, deg_sh.at[idx_v.at[CH - (CH % 3) + i]],
                        add=True)
    plsc.subcore_barrier()
    pltpu.sync_copy(deg_sh.at[pl.ds(r0, RT)], out_hbm.at[c, pl.ds(r0, RT)])


@functools.partial(
    pl.kernel,
    out_type=jax.ShapeDtypeStruct((NC, NP, D), jnp.float32),
    mesh=_mesh,
    scratch_types=[
        pltpu.VMEM((CH * K,), jnp.int32),
        pltpu.VMEM((B, K), jnp.int32),
        pltpu.VMEM((K, D), jnp.float32),
        pltpu.VMEM((K, D), jnp.float32),
        pltpu.VMEM_SHARED((NP, D), jnp.float32),
        pltpu.SemaphoreType.DMA,
        pltpu.SemaphoreType.DMA,
    ],
)
def _agg_kernel(y_hbm, src_hbm, dst_hbm, zeros_hbm, out_hbm,
                src_v, dst_v, rows0_v, rows1_v, agg_sh, sem0, sem1):
    c = lax.axis_index("c")
    s = lax.axis_index("s")
    wid = s * NC + c
    r0 = s * RT

    pltpu.sync_copy(src_hbm.at[wid], src_v)

    bufs = (rows0_v, rows1_v)
    sems = (sem0, sem1)

    def issue2(j0):
        return [pltpu.async_copy(
                    y_hbm.at[src_v.at[pl.ds((j0 + i) * K, K)]], bufs[i], sems[i])
                for i in range(2)]

    def superstep(j0, g, load_dst):
        if load_dst:
            pltpu.sync_copy(dst_hbm.at[wid, pl.ds(j0, B)], dst_v)
        for i in range(B):
            b = i % 2
            g[b].wait()
            pltpu.sync_copy(bufs[b], agg_sh.at[dst_v.at[i]], add=True)
            if i + 2 < B:
                g[b] = pltpu.async_copy(
                    y_hbm.at[src_v.at[pl.ds((j0 + i + 2) * K, K)]],
                    bufs[b], sems[b])

    g_pre = issue2(0)
    pltpu.sync_copy(dst_hbm.at[wid, pl.ds(0, B)], dst_v)

    @pl.when(c == 0)
    def _():
        pltpu.sync_copy(y_hbm.at[pl.ds(r0, RT)], agg_sh.at[pl.ds(r0, RT)])

    @pl.when(c != 0)
    def _():
        pltpu.sync_copy(zeros_hbm, agg_sh.at[pl.ds(r0, RT)])

    plsc.subcore_barrier()

    superstep(0, g_pre, load_dst=False)

    def body(it, carry):
        j0 = it * B
        superstep(j0, issue2(j0), load_dst=True)
        return carry

    lax.fori_loop(1, CH // B, body, 0, unroll=False)
    plsc.subcore_barrier()
    pltpu.sync_copy(agg_sh.at[pl.ds(r0, RT)], out_hbm.at[c, pl.ds(r0, RT)])


def _dinv_of(deg_ref):
    deg = deg_ref[0, :] + deg_ref[1, :] + 1.0
    return lax.rsqrt(deg)[:, None]


def _tc_in_body(x_ref, w_ref, deg_ref, y_ref, dinv_ref):
    dinv = _dinv_of(deg_ref)
    dinv_ref[...] = dinv
    y = jnp.dot(x_ref[...], w_ref[...], preferred_element_type=jnp.float32)
    y_ref[...] = y * dinv


def _tc_mid_body(p_ref, dinv_ref, b_ref, w_ref, y_ref):
    dinv = dinv_ref[...]
    h = jnp.maximum((p_ref[0] + p_ref[1]) * dinv + b_ref[...][None, :], 0.0)
    y_ref[...] = jnp.dot(h, w_ref[...], preferred_element_type=jnp.float32) * dinv


def _tc_out_body(p_ref, dinv_ref, b_ref, o_ref):
    o_ref[...] = jnp.maximum(
        (p_ref[0] + p_ref[1]) * dinv_ref[...] + b_ref[...][None, :], 0.0)


RO = 1000
GO = N // RO

_xspec = pl.BlockSpec((R, D), lambda i: (i, 0))
_wspec = pl.BlockSpec((D, D), lambda i: (0, 0))
_dspec = pl.BlockSpec((2, R), lambda i: (0, i))
_pspec = pl.BlockSpec((2, R, D), lambda i: (0, i, 0))
_bspec = pl.BlockSpec((D,), lambda i: (0,))
_ospec = pl.BlockSpec((R, D), lambda i: (i, 0))
_oshape = jax.ShapeDtypeStruct((NP, D), jnp.float32)

_vspec = pl.BlockSpec((R, 1), lambda i: (i, 0))
_vshape = jax.ShapeDtypeStruct((NP, 1), jnp.float32)

_tc_in = pl.pallas_call(
    _tc_in_body, grid=(G,),
    in_specs=[_xspec, _wspec, _dspec],
    out_specs=[_ospec, _vspec], out_shape=[_oshape, _vshape])
_tc_mid = pl.pallas_call(
    _tc_mid_body, grid=(G,),
    in_specs=[_pspec, _vspec, _bspec, _wspec], out_specs=_ospec, out_shape=_oshape)
_tc_out = pl.pallas_call(
    _tc_out_body, grid=(GO,),
    in_specs=[pl.BlockSpec((2, RO, D), lambda i: (0, i, 0)),
              pl.BlockSpec((RO, 1), lambda i: (i, 0)),
              _bspec],
    out_specs=pl.BlockSpec((RO, D), lambda i: (i, 0)),
    out_shape=jax.ShapeDtypeStruct((N, D), jnp.float32))


def kernel(x, edge_index, W1, b1, W2, b2):
    pad_rows = jnp.arange(EPAD - E, dtype=jnp.int32) % (NP - N)
    src = jnp.concatenate([edge_index[0], pad_rows]).reshape(NW, CH * K)
    dst = jnp.concatenate([edge_index[1], N + pad_rows]).reshape(NW, CH, K)
    zrows = jnp.zeros((RT, D), jnp.float32)

    degp = _deg_kernel(dst)
    y1, dinv = _tc_in(x, W1, degp)
    p1 = _agg_kernel(y1, src, dst, zrows)
    y2 = _tc_mid(p1, dinv, b1, W2)
    p2 = _agg_kernel(y2, src, dst, zrows)
    return _tc_out(p2, dinv, b2)
